# Initial kernel scaffold; baseline (speedup 1.0000x reference)
import functools

import jax
import jax.numpy as jnp
from jax import lax
from jax.experimental import pallas as pl
from jax.experimental.pallas import tpu as pltpu
from jax.experimental.pallas import tpu_sc as plsc

N = 10000
E = 320000
D = 128
NCLS = 16

_NUM_CORES = 2
_NUM_SUBCORES = 16
_NW = _NUM_CORES * _NUM_SUBCORES
_EPT = E // _NW
_CHUNK = 80
_NCHUNK = _EPT // _CHUNK
_ROWCHUNKS = N // _CHUNK
_RK = (_ROWCHUNKS + _NUM_SUBCORES - 1) // _NUM_SUBCORES


def _sc_gather_scatter(hw, src3, dst3):
    mesh = plsc.VectorSubcoreMesh(core_axis_name="c", subcore_axis_name="s")

    @functools.partial(
        pl.kernel,
        mesh=mesh,
        out_type=jax.ShapeDtypeStruct((_NUM_CORES, N, D), jnp.float32),
        scratch_types=[
            pltpu.VMEM((_NCHUNK, _CHUNK), jnp.int32),
            pltpu.VMEM((_NCHUNK, _CHUNK), jnp.int32),
            pltpu.VMEM((_CHUNK, D), jnp.float32),
            pltpu.VMEM_SHARED((N, D), jnp.float32),
            pltpu.SemaphoreType.DMA,
        ],
    )
    def body(hw_hbm, src_hbm, dst_hbm, out_hbm, src_v, dst_v, rows_v, acc, sem):
        c = lax.axis_index("c")
        s = lax.axis_index("s")
        wid = c * _NUM_SUBCORES + s

        def zero_rows(t, carry):
            rows_v[t // (D // 16), pl.ds((t % (D // 16)) * 16, 16)] = (
                jnp.zeros((16,), jnp.float32))
            return carry

        lax.fori_loop(0, _CHUNK * (D // 16), zero_rows, 0)

        def zero_acc(k, carry):
            j = s + k * _NUM_SUBCORES

            @pl.when(j < _ROWCHUNKS)
            def _():
                pltpu.sync_copy(rows_v, acc.at[pl.ds(j * _CHUNK, _CHUNK)])

            return carry

        lax.fori_loop(0, _RK, zero_acc, 0)
        plsc.subcore_barrier()

        pltpu.sync_copy(src_hbm.at[wid], src_v)
        pltpu.sync_copy(dst_hbm.at[wid], dst_v)

        def edge_chunk(j, carry):
            pltpu.async_copy(hw_hbm.at[src_v.at[j]], rows_v, sem).wait()
            pltpu.sync_copy(rows_v, acc.at[dst_v.at[j]], add=True)
            return carry

        lax.fori_loop(0, _NCHUNK, edge_chunk, 0)
        plsc.subcore_barrier()

        def dump(k, carry):
            j = s + k * _NUM_SUBCORES

            @pl.when(j < _ROWCHUNKS)
            def _():
                pltpu.sync_copy(acc.at[pl.ds(j * _CHUNK, _CHUNK)],
                                out_hbm.at[c, pl.ds(j * _CHUNK, _CHUNK)])

            return carry

        lax.fori_loop(0, _RK, dump, 0)

    return body(hw, src3, dst3)


_BR = 1000


def _tc_prep(feats, norm, w0l, t1l, w0g, t1g):
    def body(f_ref, n_ref, wl_ref, tl_ref, wg_ref, tg_ref, hw_ref, x_ref):
        wl = jnp.dot(wl_ref[...], tl_ref[...], preferred_element_type=jnp.float32)
        wg = jnp.dot(wg_ref[...], tg_ref[...], preferred_element_type=jnp.float32)
        f = f_ref[...]
        hw_ref[...] = jnp.dot(f, wl, preferred_element_type=jnp.float32) * n_ref[...]
        x_ref[...] = jnp.dot(f, wg, preferred_element_type=jnp.float32)

    dd = pl.BlockSpec((D, D), lambda i: (0, 0))
    return pl.pallas_call(
        body,
        grid=(N // _BR,),
        in_specs=[
            pl.BlockSpec((_BR, D), lambda i: (i, 0)),
            pl.BlockSpec((_BR, 1), lambda i: (i, 0)),
            dd, dd, dd, dd,
        ],
        out_specs=[pl.BlockSpec((_BR, D), lambda i: (i, 0))] * 2,
        out_shape=[jax.ShapeDtypeStruct((N, D), jnp.float32)] * 2,
        compiler_params=pltpu.CompilerParams(dimension_semantics=("parallel",)),
    )(feats, norm, w0l, t1l, w0g, t1g)


def _tc_ppmi(ppmi, x, bias):
    br, bk = 500, 2000

    def body(p_ref, x_ref, b_ref, o_ref):
        k = pl.program_id(1)

        @pl.when(k == 0)
        def _():
            o_ref[...] = jnp.zeros_like(o_ref)

        o_ref[...] += jnp.dot(p_ref[...], x_ref[...],
                              preferred_element_type=jnp.float32)

        @pl.when(k == pl.num_programs(1) - 1)
        def _():
            o_ref[...] += b_ref[...]

    return pl.pallas_call(
        body,
        grid=(N // br, N // bk),
        in_specs=[
            pl.BlockSpec((br, bk), lambda i, k: (i, k)),
            pl.BlockSpec((bk, D), lambda i, k: (k, 0)),
            pl.BlockSpec((1, D), lambda i, k: (0, 0)),
        ],
        out_specs=pl.BlockSpec((br, D), lambda i, k: (i, 0)),
        out_shape=jax.ShapeDtypeStruct((N, D), jnp.float32),
        compiler_params=pltpu.CompilerParams(
            dimension_semantics=("parallel", "arbitrary")),
    )(ppmi, x, bias)


def _tc_mid(agg, norm, b0l, w1l, t2l, y1, w1g, t2g):
    def body(a_ref, n_ref, b_ref, wl_ref, tl_ref, y_ref, wg_ref, tg_ref,
             hw_ref, x_ref):
        wl = jnp.dot(wl_ref[...], tl_ref[...], preferred_element_type=jnp.float32)
        wg = jnp.dot(wg_ref[...], tg_ref[...], preferred_element_type=jnp.float32)
        nrm = n_ref[...]
        h1 = (a_ref[0] + a_ref[1]) * nrm + b_ref[...]
        hw_ref[...] = jnp.dot(h1, wl, preferred_element_type=jnp.float32) * nrm
        x_ref[...] = jnp.dot(y_ref[...], wg, preferred_element_type=jnp.float32)

    dd = pl.BlockSpec((D, D), lambda i: (0, 0))
    return pl.pallas_call(
        body,
        grid=(N // _BR,),
        in_specs=[
            pl.BlockSpec((_NUM_CORES, _BR, D), lambda i: (0, i, 0)),
            pl.BlockSpec((_BR, 1), lambda i: (i, 0)),
            pl.BlockSpec((1, D), lambda i: (0, 0)),
            dd, dd,
            pl.BlockSpec((_BR, D), lambda i: (i, 0)),
            dd, dd,
        ],
        out_specs=[pl.BlockSpec((_BR, D), lambda i: (i, 0))] * 2,
        out_shape=[jax.ShapeDtypeStruct((N, D), jnp.float32)] * 2,
        compiler_params=pltpu.CompilerParams(dimension_semantics=("parallel",)),
    )(agg, norm, b0l, w1l, t2l, y1, w1g, t2g)


def _tc_fuse(agg, norm, b1l, y2, wal, wag, wc, bc):
    def body(a_ref, n_ref, b_ref, y_ref, wal_ref, wag_ref, wc_ref, bc_ref,
             o_ref):
        hl = (a_ref[0] + a_ref[1]) * n_ref[...] + b_ref[...]
        hg = y_ref[...]
        logits = (jnp.dot(hl, wal_ref[...], preferred_element_type=jnp.float32)
                  + jnp.dot(hg, wag_ref[...], preferred_element_type=jnp.float32))
        m = jnp.max(logits, axis=1, keepdims=True)
        e = jnp.exp(logits - m)
        a = e / jnp.sum(e, axis=1, keepdims=True)
        z = a[:, 0:1] * hl + a[:, 1:2] * hg
        o_ref[...] = (jnp.dot(z, wc_ref[...], preferred_element_type=jnp.float32)
                      + bc_ref[...])

    return pl.pallas_call(
        body,
        grid=(N // _BR,),
        in_specs=[
            pl.BlockSpec((_NUM_CORES, _BR, D), lambda i: (0, i, 0)),
            pl.BlockSpec((_BR, 1), lambda i: (i, 0)),
            pl.BlockSpec((1, D), lambda i: (0, 0)),
            pl.BlockSpec((_BR, D), lambda i: (i, 0)),
            pl.BlockSpec((D, 2), lambda i: (0, 0)),
            pl.BlockSpec((D, 2), lambda i: (0, 0)),
            pl.BlockSpec((D, NCLS), lambda i: (0, 0)),
            pl.BlockSpec((1, NCLS), lambda i: (0, 0)),
        ],
        out_specs=pl.BlockSpec((_BR, NCLS), lambda i: (i, 0)),
        out_shape=jax.ShapeDtypeStruct((N, NCLS), jnp.float32),
        compiler_params=pltpu.CompilerParams(dimension_semantics=("parallel",)),
    )(agg, norm, b1l, y2, wal, wag, wc, bc)


def kernel(feats, edge_index, norm, tao_1_L, tao_2_L, tao_1_G, tao_2_G, PPMI,
           w0L, b0L, w1L, b1L, w0G, b0G, w1G, b1G, W_a, W_c, b_c):
    src3 = edge_index[0].reshape(_NW, _NCHUNK, _CHUNK)
    dst3 = edge_index[1].reshape(_NW, _NCHUNK, _CHUNK)

    hw1p, x1 = _tc_prep(feats, norm, w0L, tao_1_L, w0G, tao_1_G)
    agg1 = _sc_gather_scatter(hw1p, src3, dst3)
    y1 = _tc_ppmi(PPMI, x1, b0G.reshape(1, D))
    hw2p, x2 = _tc_mid(agg1, norm, b0L.reshape(1, D), w1L, tao_2_L,
                       y1, w1G, tao_2_G)
    agg2 = _sc_gather_scatter(hw2p, src3, dst3)
    y2 = _tc_ppmi(PPMI, x2, b1G.reshape(1, D))
    return _tc_fuse(agg2, norm, b1L.reshape(1, D), y2,
                    W_a[:D], W_a[D:], W_c, b_c.reshape(1, NCLS))


# speedup vs baseline: 9.4146x; 9.4146x over previous
import functools

import jax
import jax.numpy as jnp
from jax import lax
from jax.experimental import pallas as pl
from jax.experimental.pallas import tpu as pltpu
from jax.experimental.pallas import tpu_sc as plsc

N = 10000
E = 320000
D = 128
NCLS = 16

_NUM_CORES = 2
_NUM_SUBCORES = 16
_NW = _NUM_CORES * _NUM_SUBCORES
_EPT = E // _NW
_CHUNK = 80
_NCHUNK = _EPT // _CHUNK
_ROWCHUNKS = N // _CHUNK
_RK = (_ROWCHUNKS + _NUM_SUBCORES - 1) // _NUM_SUBCORES


def _sc_gather_scatter(hw, src3, dst3):
    mesh = plsc.VectorSubcoreMesh(core_axis_name="c", subcore_axis_name="s")

    @functools.partial(
        pl.kernel,
        mesh=mesh,
        out_type=jax.ShapeDtypeStruct((_NUM_CORES, N, D), jnp.float32),
        scratch_types=[
            pltpu.VMEM((_NCHUNK, _CHUNK), jnp.int32),
            pltpu.VMEM((_NCHUNK, _CHUNK), jnp.int32),
            pltpu.VMEM((_CHUNK, D), jnp.float32),
            pltpu.VMEM_SHARED((N, D), jnp.float32),
            pltpu.SemaphoreType.DMA,
        ],
    )
    def body(hw_hbm, src_hbm, dst_hbm, out_hbm, src_v, dst_v, rows_v, acc, sem):
        c = lax.axis_index("c")
        s = lax.axis_index("s")
        wid = c * _NUM_SUBCORES + s

        def zero_rows(t, carry):
            rows_v[t // (D // 16), pl.ds((t % (D // 16)) * 16, 16)] = (
                jnp.zeros((16,), jnp.float32))
            return carry

        lax.fori_loop(0, _CHUNK * (D // 16), zero_rows, 0)

        def zero_acc(k, carry):
            j = s + k * _NUM_SUBCORES

            @pl.when(j < _ROWCHUNKS)
            def _():
                pltpu.sync_copy(rows_v, acc.at[pl.ds(j * _CHUNK, _CHUNK)])

            return carry

        lax.fori_loop(0, _RK, zero_acc, 0)
        plsc.subcore_barrier()

        pltpu.sync_copy(src_hbm.at[wid], src_v)
        pltpu.sync_copy(dst_hbm.at[wid], dst_v)

        def edge_chunk(j, carry):
            pltpu.async_copy(hw_hbm.at[src_v.at[j]], rows_v, sem).wait()
            pltpu.sync_copy(rows_v, acc.at[dst_v.at[j]], add=True)
            return carry

        lax.fori_loop(0, _NCHUNK, edge_chunk, 0)
        plsc.subcore_barrier()

        def dump(k, carry):
            j = s + k * _NUM_SUBCORES

            @pl.when(j < _ROWCHUNKS)
            def _():
                pltpu.sync_copy(acc.at[pl.ds(j * _CHUNK, _CHUNK)],
                                out_hbm.at[c, pl.ds(j * _CHUNK, _CHUNK)])

            return carry

        lax.fori_loop(0, _RK, dump, 0)

    return body(hw, src3, dst3)


_BR = 1000


def _tc_prep(feats, norm, w0l, t1l, w0g, t1g):
    def body(f_ref, n_ref, wl_ref, tl_ref, wg_ref, tg_ref, hw_ref, x_ref):
        wl = jnp.dot(wl_ref[...], tl_ref[...], preferred_element_type=jnp.float32)
        wg = jnp.dot(wg_ref[...], tg_ref[...], preferred_element_type=jnp.float32)
        f = f_ref[...]
        hw_ref[...] = jnp.dot(f, wl, preferred_element_type=jnp.float32) * n_ref[...]
        x_ref[...] = jnp.dot(f, wg, preferred_element_type=jnp.float32)

    dd = pl.BlockSpec((D, D), lambda i: (0, 0))
    return pl.pallas_call(
        body,
        grid=(N // _BR,),
        in_specs=[
            pl.BlockSpec((_BR, D), lambda i: (i, 0)),
            pl.BlockSpec((_BR, 1), lambda i: (i, 0)),
            dd, dd, dd, dd,
        ],
        out_specs=[pl.BlockSpec((_BR, D), lambda i: (i, 0))] * 2,
        out_shape=[jax.ShapeDtypeStruct((N, D), jnp.float32)] * 2,
        compiler_params=pltpu.CompilerParams(dimension_semantics=("parallel",)),
    )(feats, norm, w0l, t1l, w0g, t1g)


def _tc_ppmi(ppmi, x, bias):
    br, bk = 400, 2048
    nk = (N + bk - 1) // bk

    def body(p_ref, x_ref, b_ref, o_ref):
        k = pl.program_id(1)

        @pl.when(k == 0)
        def _():
            o_ref[...] = jnp.zeros_like(o_ref)

        rem = N - k * bk
        rows = lax.broadcasted_iota(jnp.int32, (bk, D), 0)
        xblk = jnp.where(rows < rem, x_ref[...], 0.0)
        o_ref[...] += jnp.dot(p_ref[...], xblk,
                              preferred_element_type=jnp.float32)

        @pl.when(k == pl.num_programs(1) - 1)
        def _():
            o_ref[...] += b_ref[...]

    return pl.pallas_call(
        body,
        grid=(N // br, nk),
        in_specs=[
            pl.BlockSpec((br, bk), lambda i, k: (i, k)),
            pl.BlockSpec((bk, D), lambda i, k: (k, 0)),
            pl.BlockSpec((1, D), lambda i, k: (0, 0)),
        ],
        out_specs=pl.BlockSpec((br, D), lambda i, k: (i, 0)),
        out_shape=jax.ShapeDtypeStruct((N, D), jnp.float32),
        compiler_params=pltpu.CompilerParams(
            dimension_semantics=("parallel", "arbitrary")),
    )(ppmi, x, bias)


def _tc_mid(agg, norm, b0l, w1l, t2l, y1, w1g, t2g):
    def body(a_ref, n_ref, b_ref, wl_ref, tl_ref, y_ref, wg_ref, tg_ref,
             hw_ref, x_ref):
        wl = jnp.dot(wl_ref[...], tl_ref[...], preferred_element_type=jnp.float32)
        wg = jnp.dot(wg_ref[...], tg_ref[...], preferred_element_type=jnp.float32)
        nrm = n_ref[...]
        h1 = (a_ref[0] + a_ref[1]) * nrm + b_ref[...]
        hw_ref[...] = jnp.dot(h1, wl, preferred_element_type=jnp.float32) * nrm
        x_ref[...] = jnp.dot(y_ref[...], wg, preferred_element_type=jnp.float32)

    dd = pl.BlockSpec((D, D), lambda i: (0, 0))
    return pl.pallas_call(
        body,
        grid=(N // _BR,),
        in_specs=[
            pl.BlockSpec((_NUM_CORES, _BR, D), lambda i: (0, i, 0)),
            pl.BlockSpec((_BR, 1), lambda i: (i, 0)),
            pl.BlockSpec((1, D), lambda i: (0, 0)),
            dd, dd,
            pl.BlockSpec((_BR, D), lambda i: (i, 0)),
            dd, dd,
        ],
        out_specs=[pl.BlockSpec((_BR, D), lambda i: (i, 0))] * 2,
        out_shape=[jax.ShapeDtypeStruct((N, D), jnp.float32)] * 2,
        compiler_params=pltpu.CompilerParams(dimension_semantics=("parallel",)),
    )(agg, norm, b0l, w1l, t2l, y1, w1g, t2g)


def _tc_fuse(agg, norm, b1l, y2, wal, wag, wc, bc):
    def body(a_ref, n_ref, b_ref, y_ref, wal_ref, wag_ref, wc_ref, bc_ref,
             o_ref):
        hl = (a_ref[0] + a_ref[1]) * n_ref[...] + b_ref[...]
        hg = y_ref[...]
        logits = (jnp.dot(hl, wal_ref[...], preferred_element_type=jnp.float32)
                  + jnp.dot(hg, wag_ref[...], preferred_element_type=jnp.float32))
        m = jnp.max(logits, axis=1, keepdims=True)
        e = jnp.exp(logits - m)
        a = e / jnp.sum(e, axis=1, keepdims=True)
        z = a[:, 0:1] * hl + a[:, 1:2] * hg
        o_ref[...] = (jnp.dot(z, wc_ref[...], preferred_element_type=jnp.float32)
                      + bc_ref[...])

    return pl.pallas_call(
        body,
        grid=(N // _BR,),
        in_specs=[
            pl.BlockSpec((_NUM_CORES, _BR, D), lambda i: (0, i, 0)),
            pl.BlockSpec((_BR, 1), lambda i: (i, 0)),
            pl.BlockSpec((1, D), lambda i: (0, 0)),
            pl.BlockSpec((_BR, D), lambda i: (i, 0)),
            pl.BlockSpec((D, 2), lambda i: (0, 0)),
            pl.BlockSpec((D, 2), lambda i: (0, 0)),
            pl.BlockSpec((D, NCLS), lambda i: (0, 0)),
            pl.BlockSpec((1, NCLS), lambda i: (0, 0)),
        ],
        out_specs=pl.BlockSpec((_BR, NCLS), lambda i: (i, 0)),
        out_shape=jax.ShapeDtypeStruct((N, NCLS), jnp.float32),
        compiler_params=pltpu.CompilerParams(dimension_semantics=("parallel",)),
    )(agg, norm, b1l, y2, wal, wag, wc, bc)


def kernel(feats, edge_index, norm, tao_1_L, tao_2_L, tao_1_G, tao_2_G, PPMI,
           w0L, b0L, w1L, b1L, w0G, b0G, w1G, b1G, W_a, W_c, b_c):
    src3 = edge_index[0].reshape(_NW, _NCHUNK, _CHUNK)
    dst3 = edge_index[1].reshape(_NW, _NCHUNK, _CHUNK)

    hw1p, x1 = _tc_prep(feats, norm, w0L, tao_1_L, w0G, tao_1_G)
    agg1 = _sc_gather_scatter(hw1p, src3, dst3)
    y1 = _tc_ppmi(PPMI, x1, b0G.reshape(1, D))
    hw2p, x2 = _tc_mid(agg1, norm, b0L.reshape(1, D), w1L, tao_2_L,
                       y1, w1G, tao_2_G)
    agg2 = _sc_gather_scatter(hw2p, src3, dst3)
    y2 = _tc_ppmi(PPMI, x2, b1G.reshape(1, D))
    return _tc_fuse(agg2, norm, b1L.reshape(1, D), y2,
                    W_a[:D], W_a[D:], W_c, b_c.reshape(1, NCLS))


# speedup vs baseline: 11.7206x; 1.2449x over previous
import functools

import jax
import jax.numpy as jnp
from jax import lax
from jax.experimental import pallas as pl
from jax.experimental.pallas import tpu as pltpu
from jax.experimental.pallas import tpu_sc as plsc

N = 10000
E = 320000
D = 128
NCLS = 16

_NUM_CORES = 2
_NUM_SUBCORES = 16
_NW = _NUM_CORES * _NUM_SUBCORES
_EPT = E // _NW
_CHUNK = 40
_NBUF = 5
_GPS = 5
_SLABCH = _NBUF * _GPS
_SLABPAD = 32
_NSG = _EPT // (_SLABCH * _CHUNK)
_ZSLAB = _NBUF * _CHUNK
_NSLAB = N // _ZSLAB


@functools.cache
def _sc_gather_scatter_kernel():
    mesh = plsc.VectorSubcoreMesh(core_axis_name="c", subcore_axis_name="s")

    @functools.partial(
        pl.kernel,
        mesh=mesh,
        out_type=jax.ShapeDtypeStruct((_NUM_CORES, N, D), jnp.float32),
        scratch_types=[
            pltpu.VMEM((2 * _SLABPAD, _CHUNK), jnp.int32),
            pltpu.VMEM((2 * _SLABPAD, _CHUNK), jnp.int32),
            pltpu.VMEM((_NBUF * _CHUNK, D), jnp.float32),
            pltpu.VMEM_SHARED((N, D), jnp.float32),
            pltpu.SemaphoreType.DMA,
            pltpu.SemaphoreType.DMA,
            pltpu.SemaphoreType.DMA,
        ],
    )
    def body(hw_hbm, src_hbm, dst_hbm, out_hbm, src_v, dst_v, rows_v, acc,
             gsem, ssem, isem):
        c = lax.axis_index("c")
        s = lax.axis_index("s")
        wid = c * _NUM_SUBCORES + s

        def zero_rows(t, carry):
            rows_v[t // (D // 16), pl.ds((t % (D // 16)) * 16, 16)] = (
                jnp.zeros((16,), jnp.float32))
            return carry

        lax.fori_loop(0, _ZSLAB * (D // 16), zero_rows, 0)

        def zero_acc(k, carry):
            j = s + k * _NUM_SUBCORES

            @pl.when(j < _NSLAB)
            def _():
                pltpu.sync_copy(rows_v, acc.at[pl.ds(j * _ZSLAB, _ZSLAB)])

            return carry

        lax.fori_loop(0, (_NSLAB + _NUM_SUBCORES - 1) // _NUM_SUBCORES,
                      zero_acc, 0)
        plsc.subcore_barrier()

        ebase = wid * _NSG
        pltpu.async_copy(src_hbm.at[ebase], src_v.at[pl.ds(0, _SLABPAD)],
                         isem).wait()
        pltpu.async_copy(dst_hbm.at[ebase], dst_v.at[pl.ds(0, _SLABPAD)],
                         isem).wait()

        def slab_loop(sg, carry):
            par = lax.rem(sg, 2)
            ibase = par * _SLABPAD
            nbase = (1 - par) * _SLABPAD

            @pl.when(sg > 0)
            def _():
                pltpu.make_async_copy(
                    src_hbm.at[ebase], src_v.at[pl.ds(ibase, _SLABPAD)],
                    isem).wait()
                pltpu.make_async_copy(
                    dst_hbm.at[ebase], dst_v.at[pl.ds(ibase, _SLABPAD)],
                    isem).wait()

            @pl.when(sg + 1 < _NSG)
            def _():
                pltpu.async_copy(src_hbm.at[ebase + sg + 1],
                                 src_v.at[pl.ds(nbase, _SLABPAD)], isem)
                pltpu.async_copy(dst_hbm.at[ebase + sg + 1],
                                 dst_v.at[pl.ds(nbase, _SLABPAD)], isem)

            def group(g, carry2):
                rowbase = ibase + g * _NBUF
                gh = [pltpu.async_copy(hw_hbm.at[src_v.at[rowbase + b]],
                                       rows_v.at[pl.ds(b * _CHUNK, _CHUNK)],
                                       gsem)
                      for b in range(_NBUF)]
                for h in gh:
                    h.wait()
                sh = [pltpu.async_copy(rows_v.at[pl.ds(b * _CHUNK, _CHUNK)],
                                       acc.at[dst_v.at[rowbase + b]],
                                       ssem, add=True)
                      for b in range(_NBUF)]
                for h in sh:
                    h.wait()
                return carry2

            lax.fori_loop(0, _GPS, group, 0)
            return carry

        lax.fori_loop(0, _NSG, slab_loop, 0)
        plsc.subcore_barrier()

        def dump(k, carry):
            j = s + k * _NUM_SUBCORES

            @pl.when(j < _NSLAB)
            def _():
                pltpu.sync_copy(acc.at[pl.ds(j * _ZSLAB, _ZSLAB)],
                                out_hbm.at[c, pl.ds(j * _ZSLAB, _ZSLAB)])

            return carry

        lax.fori_loop(0, (_NSLAB + _NUM_SUBCORES - 1) // _NUM_SUBCORES,
                      dump, 0)

    return body


def _sc_gather_scatter(hw, src3, dst3):
    return _sc_gather_scatter_kernel()(hw, src3, dst3)


_BR = 1000


def _tc_prep(feats, norm, w0l, t1l, w0g, t1g):
    def body(f_ref, n_ref, wl_ref, tl_ref, wg_ref, tg_ref, hw_ref, x_ref):
        wl = jnp.dot(wl_ref[...], tl_ref[...], preferred_element_type=jnp.float32)
        wg = jnp.dot(wg_ref[...], tg_ref[...], preferred_element_type=jnp.float32)
        f = f_ref[...]
        hw_ref[...] = jnp.dot(f, wl, preferred_element_type=jnp.float32) * n_ref[...]
        x_ref[...] = jnp.dot(f, wg, preferred_element_type=jnp.float32)

    dd = pl.BlockSpec((D, D), lambda i: (0, 0))
    return pl.pallas_call(
        body,
        grid=(N // _BR,),
        in_specs=[
            pl.BlockSpec((_BR, D), lambda i: (i, 0)),
            pl.BlockSpec((_BR, 1), lambda i: (i, 0)),
            dd, dd, dd, dd,
        ],
        out_specs=[pl.BlockSpec((_BR, D), lambda i: (i, 0))] * 2,
        out_shape=[jax.ShapeDtypeStruct((N, D), jnp.float32)] * 2,
        compiler_params=pltpu.CompilerParams(dimension_semantics=("parallel",)),
    )(feats, norm, w0l, t1l, w0g, t1g)


def _tc_ppmi(ppmi, x, bias):
    br = 400

    def body(p_ref, x_ref, b_ref, o_ref):
        o_ref[...] = (jnp.dot(p_ref[...], x_ref[...],
                              preferred_element_type=jnp.float32)
                      + b_ref[...])

    return pl.pallas_call(
        body,
        grid=(N // br,),
        in_specs=[
            pl.BlockSpec((br, N), lambda i: (i, 0)),
            pl.BlockSpec((N, D), lambda i: (0, 0)),
            pl.BlockSpec((1, D), lambda i: (0, 0)),
        ],
        out_specs=pl.BlockSpec((br, D), lambda i: (i, 0)),
        out_shape=jax.ShapeDtypeStruct((N, D), jnp.float32),
        compiler_params=pltpu.CompilerParams(
            dimension_semantics=("arbitrary",)),
    )(ppmi, x, bias)


def _tc_mid(agg, norm, b0l, w1l, t2l, y1, w1g, t2g):
    def body(a_ref, n_ref, b_ref, wl_ref, tl_ref, y_ref, wg_ref, tg_ref,
             hw_ref, x_ref):
        wl = jnp.dot(wl_ref[...], tl_ref[...], preferred_element_type=jnp.float32)
        wg = jnp.dot(wg_ref[...], tg_ref[...], preferred_element_type=jnp.float32)
        nrm = n_ref[...]
        h1 = (a_ref[0] + a_ref[1]) * nrm + b_ref[...]
        hw_ref[...] = jnp.dot(h1, wl, preferred_element_type=jnp.float32) * nrm
        x_ref[...] = jnp.dot(y_ref[...], wg, preferred_element_type=jnp.float32)

    dd = pl.BlockSpec((D, D), lambda i: (0, 0))
    return pl.pallas_call(
        body,
        grid=(N // _BR,),
        in_specs=[
            pl.BlockSpec((_NUM_CORES, _BR, D), lambda i: (0, i, 0)),
            pl.BlockSpec((_BR, 1), lambda i: (i, 0)),
            pl.BlockSpec((1, D), lambda i: (0, 0)),
            dd, dd,
            pl.BlockSpec((_BR, D), lambda i: (i, 0)),
            dd, dd,
        ],
        out_specs=[pl.BlockSpec((_BR, D), lambda i: (i, 0))] * 2,
        out_shape=[jax.ShapeDtypeStruct((N, D), jnp.float32)] * 2,
        compiler_params=pltpu.CompilerParams(dimension_semantics=("parallel",)),
    )(agg, norm, b0l, w1l, t2l, y1, w1g, t2g)


def _tc_fuse(agg, norm, b1l, y2, wal, wag, wc, bc):
    def body(a_ref, n_ref, b_ref, y_ref, wal_ref, wag_ref, wc_ref, bc_ref,
             o_ref):
        hl = (a_ref[0] + a_ref[1]) * n_ref[...] + b_ref[...]
        hg = y_ref[...]
        logits = (jnp.dot(hl, wal_ref[...], preferred_element_type=jnp.float32)
                  + jnp.dot(hg, wag_ref[...], preferred_element_type=jnp.float32))
        m = jnp.max(logits, axis=1, keepdims=True)
        e = jnp.exp(logits - m)
        a = e / jnp.sum(e, axis=1, keepdims=True)
        z = a[:, 0:1] * hl + a[:, 1:2] * hg
        o_ref[...] = (jnp.dot(z, wc_ref[...], preferred_element_type=jnp.float32)
                      + bc_ref[...])

    return pl.pallas_call(
        body,
        grid=(N // _BR,),
        in_specs=[
            pl.BlockSpec((_NUM_CORES, _BR, D), lambda i: (0, i, 0)),
            pl.BlockSpec((_BR, 1), lambda i: (i, 0)),
            pl.BlockSpec((1, D), lambda i: (0, 0)),
            pl.BlockSpec((_BR, D), lambda i: (i, 0)),
            pl.BlockSpec((D, 2), lambda i: (0, 0)),
            pl.BlockSpec((D, 2), lambda i: (0, 0)),
            pl.BlockSpec((D, NCLS), lambda i: (0, 0)),
            pl.BlockSpec((1, NCLS), lambda i: (0, 0)),
        ],
        out_specs=pl.BlockSpec((_BR, NCLS), lambda i: (i, 0)),
        out_shape=jax.ShapeDtypeStruct((N, NCLS), jnp.float32),
        compiler_params=pltpu.CompilerParams(dimension_semantics=("parallel",)),
    )(agg, norm, b1l, y2, wal, wag, wc, bc)


def kernel(feats, edge_index, norm, tao_1_L, tao_2_L, tao_1_G, tao_2_G, PPMI,
           w0L, b0L, w1L, b1L, w0G, b0G, w1G, b1G, W_a, W_c, b_c):
    pad = ((0, 0), (0, _SLABPAD - _SLABCH), (0, 0))
    src3 = jnp.pad(edge_index[0].reshape(_NW * _NSG, _SLABCH, _CHUNK), pad)
    dst3 = jnp.pad(edge_index[1].reshape(_NW * _NSG, _SLABCH, _CHUNK), pad)

    hw1p, x1 = _tc_prep(feats, norm, w0L, tao_1_L, w0G, tao_1_G)
    agg1 = _sc_gather_scatter(hw1p, src3, dst3)
    y1 = _tc_ppmi(PPMI, x1, b0G.reshape(1, D))
    hw2p, x2 = _tc_mid(agg1, norm, b0L.reshape(1, D), w1L, tao_2_L,
                       y1, w1G, tao_2_G)
    agg2 = _sc_gather_scatter(hw2p, src3, dst3)
    y2 = _tc_ppmi(PPMI, x2, b1G.reshape(1, D))
    return _tc_fuse(agg2, norm, b1L.reshape(1, D), y2,
                    W_a[:D], W_a[D:], W_c, b_c.reshape(1, NCLS))
